# baseline (speedup 1.0000x reference)
import functools

import jax
import jax.numpy as jnp
from jax import lax
from jax.experimental import pallas as pl
from jax.experimental.pallas import tpu as pltpu
from jax.experimental.pallas import tpu_sc as plsc

_R = 64
_L = 16


def kernel(encoded_tokens, pos_table):
    batch, num_tokens, embed = encoded_tokens.shape
    n_rows = batch * num_tokens
    tokens2d = encoded_tokens.reshape(n_rows, embed)

    info = plsc.get_sparse_core_info()
    nc, ns = info.num_cores, info.num_subcores
    nw = nc * ns
    rows_pw = n_rows // nw
    n_chunks = rows_pw // _R
    chunk_elems = _R * embed
    assert rows_pw % _R == 0 and num_tokens % rows_pw == 0

    mesh = plsc.VectorSubcoreMesh(core_axis_name="c", subcore_axis_name="s")

    @functools.partial(
        pl.kernel,
        mesh=mesh,
        out_type=jax.ShapeDtypeStruct((n_rows, embed), jnp.float32),
        scratch_types=[
            pltpu.VMEM((_R, embed), jnp.float32),
            pltpu.VMEM((_R, embed), jnp.float32),
            pltpu.SemaphoreType.DMA,
        ],
    )
    def sc_add(tok_hbm, pos_hbm, out_hbm, buf_v, pos_v, sem):
        wid = lax.axis_index("s") * nc + lax.axis_index("c")
        row0 = wid * rows_pw
        t0 = lax.rem(row0, num_tokens)

        def chunk_body(g, carry):
            base = row0 + g * _R
            tb = t0 + g * _R
            cp = pltpu.async_copy(tok_hbm.at[pl.ds(base, _R)], buf_v, sem)
            pltpu.sync_copy(pos_hbm.at[pl.ds(tb, _R)], pos_v)
            cp.wait()

            def add_row(r, c):
                for i in range(embed // _L):
                    plsc.addupdate(
                        buf_v.at[r, pl.ds(i * _L, _L)],
                        pos_v[r, pl.ds(i * _L, _L)],
                    )
                return c

            lax.fori_loop(0, _R, add_row, 0)
            pltpu.sync_copy(buf_v, out_hbm.at[pl.ds(base, _R)])
            return carry

        lax.fori_loop(0, n_chunks, chunk_body, 0)

    out = sc_add(tokens2d, pos_table)
    return out.reshape(batch, num_tokens, embed)
